# baseline (speedup 1.0000x reference)
import functools

import jax
import jax.numpy as jnp
from jax import lax
from jax.experimental import pallas as pl
from jax.experimental.pallas import tpu as pltpu
from jax.experimental.pallas import tpu_sc as plsc


def kernel(etypes, table):
    E = etypes.shape[0]
    V, D = table.shape

    info = plsc.get_sparse_core_info()
    NC, NS = info.num_cores, info.num_subcores
    NW = NC * NS

    CH = 128
    EPC = 2 * CH
    n_chunks = E // EPC
    assert n_chunks * EPC == E
    n_even = n_chunks // NW
    n_extra = n_chunks - n_even * NW

    mesh = plsc.VectorSubcoreMesh(core_axis_name="c", subcore_axis_name="s")

    @functools.partial(
        pl.kernel,
        mesh=mesh,
        compiler_params=pltpu.CompilerParams(
            use_tc_tiling_on_sc=False, needs_layout_passes=False
        ),
        out_type=jax.ShapeDtypeStruct((E // 2, 2 * D), jnp.float32),
        scratch_types=[
            pltpu.VMEM((V, D), jnp.float32),
            pltpu.VMEM_SHARED((V * V, 2 * D), jnp.float32),
            pltpu.VMEM((EPC,), jnp.int32),
            pltpu.VMEM((CH,), jnp.int32),
            pltpu.VMEM((CH, 2 * D), jnp.float32),
            pltpu.SemaphoreType.DMA,
        ],
    )
    def emb_kernel(
        etypes_hbm, table_hbm, out_hbm, tab_v, pairs_sp, eidx_v, pidx_v,
        rows_v, sem,
    ):
        sid = lax.axis_index("s")
        wid = sid * NC + lax.axis_index("c")

        pltpu.sync_copy(table_hbm, tab_v)
        for j in range(V):
            p = sid * V + j
            pltpu.sync_copy(tab_v.at[sid], pairs_sp.at[p, pl.ds(0, D)])
            pltpu.sync_copy(tab_v.at[j], pairs_sp.at[p, pl.ds(D, D)])
        plsc.subcore_barrier()

        lanes = lax.iota(jnp.int32, 16)

        def do_chunk(c):
            pltpu.sync_copy(etypes_hbm.at[pl.ds(c * EPC, EPC)], eidx_v)
            for g in range(CH // 16):
                b = lanes * 2 + 32 * g
                ev = plsc.load_gather(eidx_v, [b])
                od = plsc.load_gather(eidx_v, [b + 1])
                pidx_v[pl.ds(g * 16, 16)] = ev * V + od
            pltpu.async_copy(pairs_sp.at[pidx_v], rows_v, sem).wait()
            pltpu.sync_copy(rows_v, out_hbm.at[pl.ds(c * CH, CH)])

        def body(t, carry):
            do_chunk(wid + NW * t)
            return carry

        n_w = jnp.where(wid < n_extra, n_even + 1, n_even)
        lax.fori_loop(0, n_w, body, 0)

    return emb_kernel(etypes, table).reshape(E, D)
